# Initial kernel scaffold; baseline (speedup 1.0000x reference)
import jax
import jax.numpy as jnp
from jax.experimental import pallas as pl


def kernel(x, edge_index, batch, emb, W1_0, b1_0, g1_0, be1_0, W2_0, b2_0, eps_0, bng_0, bnb_0, W1_1, b1_1, g1_1, be1_1, W2_1, b2_1, eps_1, bng_1, bnb_1, W1_2, b1_2, g1_2, be1_2, W2_2, b2_2, eps_2, bng_2, bnb_2, W1_3, b1_3, g1_3, be1_3, W2_3, b2_3, eps_3, bng_3, bnb_3, PW1, Pb1, PW2, Pb2):
    raise NotImplementedError("write your pallas kernel here")


# speedup vs baseline: 3.6066x; 3.6066x over previous
import functools

import jax
import jax.numpy as jnp
from jax import lax
from jax.experimental import pallas as pl
from jax.experimental.pallas import tpu as pltpu
from jax.experimental.pallas import tpu_sc as plsc

N = 10000
E = 160000
B = 64
NODE_DIM = 256
HID = 512
OUT_DIM = 512
VOCAB = 76
NUM_LAYERS = 4

NP_ = 10240
NSUB = 16
NCORE = 2
SR = NP_ // NSUB
G = 128
EPT = 10112
CH = EPT // G
EPAD = NSUB * EPT
VPAD = 80
XCH = NP_ // NSUB // G

BN = 512
BNP = 1024
GRIDP = NP_ // BNP

INV_BN = 1.0 / (1.0 + 1e-5) ** 0.5


def _sc_mesh():
    return plsc.VectorSubcoreMesh(core_axis_name="c", subcore_axis_name="s")


def _embed_body(emb_hbm, xq_hbm, out_hbm, xv, rows, sem):
    cid = lax.axis_index("c")
    sid = lax.axis_index("s")
    pltpu.sync_copy(xq_hbm.at[cid * NSUB + sid], xv)
    for c in range(XCH):
        pltpu.async_copy(emb_hbm.at[xv.at[c]], rows, sem).wait()
        pltpu.sync_copy(
            rows, out_hbm.at[pl.ds(cid * NP_ + sid * (NP_ // NSUB) + c * G, G)])


def _sc_embed(embflat, xq):
    fn = pl.kernel(
        _embed_body,
        out_type=jax.ShapeDtypeStruct((2 * NP_, 128), jnp.float32),
        mesh=_sc_mesh(),
        scratch_types=[
            pltpu.VMEM((XCH, G), jnp.int32),
            pltpu.VMEM((G, 128), jnp.float32),
            pltpu.SemaphoreType.DMA,
        ],
    )
    return fn(embflat, xq)


def _agg_body(nq, hflat_hbm, srcq_hbm, dst_hbm, zeros_hbm, out_hbm,
              srcv, dstv, rows, acc, sem):
    cid = lax.axis_index("c")
    sid = lax.axis_index("s")
    pltpu.sync_copy(dst_hbm.at[sid], dstv)
    for qi in range(nq // NCORE):
        q = qi * NCORE + cid
        pltpu.sync_copy(srcq_hbm.at[q * NSUB + sid], srcv)
        pltpu.sync_copy(zeros_hbm, acc.at[pl.ds(sid * SR, SR)])
        plsc.subcore_barrier()

        def chunk(c, carry):
            pltpu.async_copy(hflat_hbm.at[srcv.at[c]], rows, sem).wait()
            pltpu.sync_copy(rows, acc.at[dstv.at[c]], add=True)
            return carry

        lax.fori_loop(0, CH, chunk, 0)
        plsc.subcore_barrier()
        pltpu.sync_copy(acc.at[pl.ds(sid * SR, SR)],
                        out_hbm.at[pl.ds(q * NP_ + sid * SR, SR)])


def _sc_agg(nq, hflat, srcq, dst3, zeros):
    fn = pl.kernel(
        functools.partial(_agg_body, nq),
        out_type=jax.ShapeDtypeStruct((nq * NP_, 128), jnp.float32),
        mesh=_sc_mesh(),
        scratch_types=[
            pltpu.VMEM((CH, G), jnp.int32),
            pltpu.VMEM((CH, G), jnp.int32),
            pltpu.VMEM((G, 128), jnp.float32),
            pltpu.VMEM_SHARED((NP_, 128), jnp.float32),
            pltpu.SemaphoreType.DMA,
        ],
    )
    return fn(hflat, srcq, dst3, zeros)


def _mlp_body(qin, h_ref, a_ref, eps_ref, w1_ref, b1_ref, g1_ref, be1_ref,
              w2_ref, b2_ref, bng_ref, bnb_ref, out_ref):
    h = jnp.concatenate([h_ref[qq] for qq in range(qin)], axis=1)
    a = jnp.concatenate([a_ref[qq] for qq in range(qin)], axis=1)
    z = (1.0 + eps_ref[0, 0]) * h + a
    z = jnp.dot(z, w1_ref[...], preferred_element_type=jnp.float32) + b1_ref[...]
    z = jnp.maximum(z * (g1_ref[...] * INV_BN) + be1_ref[...], 0.0)
    z = jnp.dot(z, w2_ref[...], preferred_element_type=jnp.float32) + b2_ref[...]
    z = jnp.maximum(z * (bng_ref[...] * INV_BN) + bnb_ref[...], 0.0)
    for qq in range(4):
        out_ref[qq] = z[:, qq * 128:(qq + 1) * 128]


def _tc_mlp(qin, din, h4, a4, eps, w1, b1, g1, be1, w2, b2, bng, bnb):
    grid = NP_ // BN
    vec = lambda: pl.BlockSpec((1, HID), lambda i: (0, 0))
    return pl.pallas_call(
        functools.partial(_mlp_body, qin),
        grid=(grid,),
        in_specs=[
            pl.BlockSpec((qin, BN, 128), lambda i: (0, i, 0)),
            pl.BlockSpec((qin, BN, 128), lambda i: (0, i, 0)),
            pl.BlockSpec((1, 1), lambda i: (0, 0)),
            pl.BlockSpec((din, HID), lambda i: (0, 0)),
            vec(), vec(), vec(),
            pl.BlockSpec((HID, HID), lambda i: (0, 0)),
            vec(), vec(), vec(),
        ],
        out_specs=pl.BlockSpec((4, BN, 128), lambda i: (0, i, 0)),
        out_shape=jax.ShapeDtypeStruct((4, NP_, 128), jnp.float32),
    )(h4, a4, eps, w1, b1, g1, be1, w2, b2, bng, bnb)


def _pool_body(h_ref, b_ref, pw1_ref, pb1_ref, pw2_ref, pb2_ref, out_ref,
               sums_ref, cnt_ref):
    i = pl.program_id(0)

    @pl.when(i == 0)
    def _init():
        sums_ref[...] = jnp.zeros((128, HID), jnp.float32)
        cnt_ref[...] = jnp.zeros((128, 128), jnp.float32)

    h = jnp.concatenate([h_ref[qq] for qq in range(4)], axis=1)
    brow = b_ref[0]
    oh = (lax.broadcasted_iota(jnp.int32, (128, BNP), 0) == brow
          ).astype(jnp.float32)
    sums_ref[...] += jnp.dot(oh, h, preferred_element_type=jnp.float32)
    cnt_ref[...] += jnp.dot(oh, jnp.ones((BNP, 128), jnp.float32),
                            preferred_element_type=jnp.float32)

    @pl.when(i == GRIDP - 1)
    def _final():
        r = 1.0 / jnp.maximum(cnt_ref[...], 1.0)
        hg = sums_ref[...] * jnp.concatenate([r, r, r, r], axis=1)
        z = jnp.maximum(
            jnp.dot(hg, pw1_ref[...], preferred_element_type=jnp.float32)
            + pb1_ref[...], 0.0)
        res = (jnp.dot(z, pw2_ref[...], preferred_element_type=jnp.float32)
               + pb2_ref[...])
        out_ref[...] = res[:B, :]


def _tc_pool(h4, batch3, pw1, pb1, pw2, pb2):
    return pl.pallas_call(
        _pool_body,
        grid=(GRIDP,),
        in_specs=[
            pl.BlockSpec((4, BNP, 128), lambda i: (0, i, 0)),
            pl.BlockSpec((1, 1, BNP), lambda i: (i, 0, 0)),
            pl.BlockSpec((HID, HID), lambda i: (0, 0)),
            pl.BlockSpec((1, HID), lambda i: (0, 0)),
            pl.BlockSpec((HID, OUT_DIM), lambda i: (0, 0)),
            pl.BlockSpec((1, OUT_DIM), lambda i: (0, 0)),
        ],
        out_specs=pl.BlockSpec((B, OUT_DIM), lambda i: (0, 0)),
        out_shape=jax.ShapeDtypeStruct((B, OUT_DIM), jnp.float32),
        scratch_shapes=[
            pltpu.VMEM((128, HID), jnp.float32),
            pltpu.VMEM((128, 128), jnp.float32),
        ],
    )(h4, batch3, pw1, pb1, pw2, pb2)


def kernel(x, edge_index, batch, emb,
           W1_0, b1_0, g1_0, be1_0, W2_0, b2_0, eps_0, bng_0, bnb_0,
           W1_1, b1_1, g1_1, be1_1, W2_1, b2_1, eps_1, bng_1, bnb_1,
           W1_2, b1_2, g1_2, be1_2, W2_2, b2_2, eps_2, bng_2, bnb_2,
           W1_3, b1_3, g1_3, be1_3, W2_3, b2_3, eps_3, bng_3, bnb_3,
           PW1, Pb1, PW2, Pb2):
    f32 = jnp.float32

    x_pad = jnp.pad(x, (0, NP_ - N))
    xq3 = x_pad.reshape(NSUB, XCH, G)
    xq = jnp.concatenate([xq3 + q * VPAD for q in range(2)], axis=0)

    src = jnp.pad(edge_index[0], (0, EPAD - E)).reshape(NSUB, CH, G)
    dst3 = jnp.pad(edge_index[1], (0, EPAD - E),
                   constant_values=N).reshape(NSUB, CH, G)
    src2 = jnp.concatenate([src + q * NP_ for q in range(2)], axis=0)
    src4 = jnp.concatenate([src + q * NP_ for q in range(4)], axis=0)

    batch3 = jnp.pad(batch, (0, NP_ - N), constant_values=B).reshape(
        GRIDP, 1, BNP)
    zeros = jnp.zeros((SR, 128), f32)

    emb_pad = jnp.pad(emb, ((0, VPAD - VOCAB), (0, 0)))
    embflat = jnp.concatenate(
        [emb_pad[:, q * 128:(q + 1) * 128] for q in range(2)], axis=0)

    layers = [
        (eps_0, W1_0, b1_0, g1_0, be1_0, W2_0, b2_0, bng_0, bnb_0),
        (eps_1, W1_1, b1_1, g1_1, be1_1, W2_1, b2_1, bng_1, bnb_1),
        (eps_2, W1_2, b1_2, g1_2, be1_2, W2_2, b2_2, bng_2, bnb_2),
        (eps_3, W1_3, b1_3, g1_3, be1_3, W2_3, b2_3, bng_3, bnb_3),
    ]

    hflat = _sc_embed(embflat, xq)

    for l in range(NUM_LAYERS):
        qin = 2 if l == 0 else 4
        din = NODE_DIM if l == 0 else HID
        srcq = src2 if l == 0 else src4
        eps, w1, b1, g1, be1, w2, b2, bng, bnb = layers[l]
        aggflat = _sc_agg(qin, hflat, srcq, dst3, zeros)
        h4 = _tc_mlp(
            qin, din,
            hflat.reshape(qin, NP_, 128), aggflat.reshape(qin, NP_, 128),
            eps.reshape(1, 1), w1, b1.reshape(1, HID), g1.reshape(1, HID),
            be1.reshape(1, HID), w2, b2.reshape(1, HID),
            bng.reshape(1, HID), bnb.reshape(1, HID))
        hflat = h4.reshape(4 * NP_, 128)

    return _tc_pool(h4, batch3, PW1, Pb1.reshape(1, HID),
                    PW2, Pb2.reshape(1, OUT_DIM))
